# baseline (speedup 1.0000x reference)
import jax
import jax.numpy as jnp
from jax import lax
from jax.experimental import pallas as pl
from jax.experimental.pallas import tpu as pltpu
from jax.experimental.pallas import tpu_sc as plsc

_C, _T, _H, _W = 3, 64, 256, 256
_S = _T // 4
_ROW = _H * _W
_HALF = _ROW // 2
_NW = 32
_ROWS_PER_W = (_C * _T) // _NW


def _sc_body(frames_ref, slow_ref, fast_ref, buf):
    wid = lax.axis_index("s") * 2 + lax.axis_index("c")
    for j in range(_ROWS_PER_W):
        r = wid + _NW * j
        t = lax.rem(r, _T)
        c = lax.div(r, _T)
        s = lax.div(t * (_S - 1) + (_T - 2), _T - 1)
        sel = lax.div(s * (_T - 1), _S - 1) == t
        slow_r = c * _S + s
        for h in range(2):
            src = frames_ref.at[r, pl.ds(h * _HALF, _HALF)]
            pltpu.sync_copy(src, buf)
            pltpu.sync_copy(buf, fast_ref.at[r, pl.ds(h * _HALF, _HALF)])

            @pl.when(sel)
            def _():
                pltpu.sync_copy(
                    buf, slow_ref.at[slow_r, pl.ds(h * _HALF, _HALF)]
                )


def kernel(frames):
    C, T, H, W = frames.shape
    flat = frames.reshape(C * T, H * W)
    mesh = plsc.VectorSubcoreMesh(core_axis_name="c", subcore_axis_name="s")
    slow, fast = pl.kernel(
        _sc_body,
        out_type=[
            jax.ShapeDtypeStruct((C * (T // 4), H * W), frames.dtype),
            jax.ShapeDtypeStruct((C * T, H * W), frames.dtype),
        ],
        mesh=mesh,
        scratch_types=[pltpu.VMEM((_HALF,), jnp.float32)],
    )(flat)
    return (
        slow.reshape(C, T // 4, H, W),
        fast.reshape(C, T, H, W),
    )


# speedup vs baseline: 1.0529x; 1.0529x over previous
import jax
import jax.numpy as jnp
from jax import lax
from jax.experimental import pallas as pl
from jax.experimental.pallas import tpu as pltpu
from jax.experimental.pallas import tpu_sc as plsc

_C, _T, _H, _W = 3, 64, 256, 256
_S = _T // 4
_ROW = _H * _W
_CHUNK = _ROW // 4
_CPR = _ROW // _CHUNK
_NW = 32
_RPW = (_C * _T) // _NW
_NCH = _RPW * _CPR
_RING = 7


def _sc_body(frames_ref, slow_ref, fast_ref, *scratch):
    bufs = scratch[:_RING]
    sin = scratch[_RING:2 * _RING]
    sout = scratch[2 * _RING:3 * _RING]
    wid = lax.axis_index("s") * 2 + lax.axis_index("c")

    def info(k):
        j, h = divmod(k, _CPR)
        r = wid + _NW * j
        t = lax.rem(r, _T)
        c = lax.div(r, _T)
        s = lax.div(t * (_S - 1) + (_T - 2), _T - 1)
        sel = lax.div(s * (_T - 1), _S - 1) == t
        return r, h * _CHUNK, sel, c * _S + s

    def in_copy(k):
        r, off, _, _ = info(k)
        return pltpu.make_async_copy(
            frames_ref.at[r, pl.ds(off, _CHUNK)], bufs[k % _RING], sin[k % _RING]
        )

    def fast_copy(k):
        r, off, _, _ = info(k)
        return pltpu.make_async_copy(
            bufs[k % _RING], fast_ref.at[r, pl.ds(off, _CHUNK)], sout[k % _RING]
        )

    def slow_copy(k):
        _, off, _, sr = info(k)
        return pltpu.make_async_copy(
            bufs[k % _RING], slow_ref.at[sr, pl.ds(off, _CHUNK)], sout[k % _RING]
        )

    def start_outs(k):
        fast_copy(k).start()

        @pl.when(info(k)[2])
        def _():
            slow_copy(k).start()

    def wait_outs(k):
        fast_copy(k).wait()

        @pl.when(info(k)[2])
        def _():
            slow_copy(k).wait()

    for k in range(4):
        in_copy(k).start()
    for k in range(_NCH):
        in_copy(k).wait()
        start_outs(k)
        if k >= 3:
            wait_outs(k - 3)
        if k + 4 < _NCH:
            in_copy(k + 4).start()
    for k in range(_NCH - 3, _NCH):
        wait_outs(k)


def kernel(frames):
    C, T, H, W = frames.shape
    flat = frames.reshape(C * T, H * W)
    mesh = plsc.VectorSubcoreMesh(core_axis_name="c", subcore_axis_name="s")
    scratch = (
        [pltpu.VMEM((_CHUNK,), jnp.float32) for _ in range(_RING)]
        + [pltpu.SemaphoreType.DMA for _ in range(2 * _RING)]
    )
    slow, fast = pl.kernel(
        _sc_body,
        out_type=[
            jax.ShapeDtypeStruct((C * (T // 4), H * W), frames.dtype),
            jax.ShapeDtypeStruct((C * T, H * W), frames.dtype),
        ],
        mesh=mesh,
        scratch_types=scratch,
    )(flat)
    return (
        slow.reshape(C, T // 4, H, W),
        fast.reshape(C, T, H, W),
    )


# speedup vs baseline: 2.5877x; 2.4576x over previous
import jax
import jax.numpy as jnp
from jax import lax
from jax.experimental import pallas as pl
from jax.experimental.pallas import tpu as pltpu
from jax.experimental.pallas import tpu_sc as plsc

_C, _T, _H, _W = 3, 64, 256, 256
_S = _T // 4
_CPP = 4
_CH = _H // _CPP
_NW = 32
_PPW = (_C * _T) // _NW
_NCH = _PPW * _CPP
_RING = 7


def _sc_body(frames_ref, slow_ref, fast_ref, *scratch):
    bufs = scratch[:_RING]
    sin = scratch[_RING:2 * _RING]
    sout = scratch[2 * _RING:3 * _RING]
    wid = lax.axis_index("s") * 2 + lax.axis_index("c")

    def info(k):
        j, h = divmod(k, _CPP)
        r = wid + _NW * j
        t = lax.rem(r, _T)
        c = lax.div(r, _T)
        s = lax.div(t * (_S - 1) + (_T - 2), _T - 1)
        sel = lax.div(s * (_T - 1), _S - 1) == t
        return c, t, h * _CH, sel, s

    def in_copy(k):
        c, t, row, _, _ = info(k)
        return pltpu.make_async_copy(
            frames_ref.at[c, t, pl.ds(row, _CH), :],
            bufs[k % _RING],
            sin[k % _RING],
        )

    def fast_copy(k):
        c, t, row, _, _ = info(k)
        return pltpu.make_async_copy(
            bufs[k % _RING],
            fast_ref.at[c, t, pl.ds(row, _CH), :],
            sout[k % _RING],
        )

    def slow_copy(k):
        c, _, row, _, s = info(k)
        return pltpu.make_async_copy(
            bufs[k % _RING],
            slow_ref.at[c, s, pl.ds(row, _CH), :],
            sout[k % _RING],
        )

    def start_outs(k):
        fast_copy(k).start()

        @pl.when(info(k)[3])
        def _():
            slow_copy(k).start()

    def wait_outs(k):
        fast_copy(k).wait()

        @pl.when(info(k)[3])
        def _():
            slow_copy(k).wait()

    for k in range(4):
        in_copy(k).start()
    for k in range(_NCH):
        in_copy(k).wait()
        start_outs(k)
        if k >= 3:
            wait_outs(k - 3)
        if k + 4 < _NCH:
            in_copy(k + 4).start()
    for k in range(_NCH - 3, _NCH):
        wait_outs(k)


def kernel(frames):
    C, T, H, W = frames.shape
    mesh = plsc.VectorSubcoreMesh(core_axis_name="c", subcore_axis_name="s")
    scratch = (
        [pltpu.VMEM((_CH, W), jnp.float32) for _ in range(_RING)]
        + [pltpu.SemaphoreType.DMA for _ in range(2 * _RING)]
    )
    slow, fast = pl.kernel(
        _sc_body,
        out_type=[
            jax.ShapeDtypeStruct((C, T // 4, H, W), frames.dtype),
            jax.ShapeDtypeStruct((C, T, H, W), frames.dtype),
        ],
        mesh=mesh,
        scratch_types=scratch,
    )(frames)
    return (slow, fast)
